# Initial kernel scaffold; baseline (speedup 1.0000x reference)
import jax
import jax.numpy as jnp
from jax.experimental import pallas as pl


def kernel(x, edge_index, W, W1, b1, W2, b2, gamma, beta):
    raise NotImplementedError("write your pallas kernel here")


# speedup vs baseline: 1.9922x; 1.9922x over previous
import functools

import jax
import jax.numpy as jnp
from jax import lax
from jax.experimental import pallas as pl
from jax.experimental.pallas import tpu as pltpu
from jax.experimental.pallas import tpu_sc as plsc

N = 10000
DIM = 128
E = 320000

NC = 2
NS = 16
NW = NC * NS

ACC_W = 144
N_PAD = 10240
ROWS_PER_SUB = N_PAD // NS
CHUNK = 40
EDGES_PER_W = E // NW
N_CHUNKS = EDGES_PER_W // CHUNK

ROW_BLK = 2000


def _node_mm_body(x_ref, wh_ref, bh_ref, a_ref, bc_ref):
    y = jnp.dot(x_ref[...], wh_ref[...], preferred_element_type=jnp.float32)
    y = y + bh_ref[...]
    a_ref[...] = y[:, :DIM]
    bc_ref[...] = y[:, DIM:]


_node_mm = pl.pallas_call(
    _node_mm_body,
    grid=(N // ROW_BLK,),
    in_specs=[
        pl.BlockSpec((ROW_BLK, DIM), lambda i: (i, 0)),
        pl.BlockSpec((DIM, 3 * DIM), lambda i: (0, 0)),
        pl.BlockSpec((1, 3 * DIM), lambda i: (0, 0)),
    ],
    out_specs=[
        pl.BlockSpec((ROW_BLK, DIM), lambda i: (i, 0)),
        pl.BlockSpec((ROW_BLK, 2 * DIM), lambda i: (i, 0)),
    ],
    out_shape=[
        jax.ShapeDtypeStruct((N, DIM), jnp.float32),
        jax.ShapeDtypeStruct((N, 2 * DIM), jnp.float32),
    ],
)


_sc_mesh = plsc.VectorSubcoreMesh(core_axis_name="c", subcore_axis_name="s")


@functools.partial(
    pl.kernel,
    out_type=jax.ShapeDtypeStruct((NC, N_PAD, ACC_W), jnp.float32),
    mesh=_sc_mesh,
    compiler_params=pltpu.CompilerParams(needs_layout_passes=False,
                                         use_tc_tiling_on_sc=False),
    scratch_types=[
        pltpu.VMEM((CHUNK,), jnp.int32),
        pltpu.VMEM((CHUNK,), jnp.int32),
        pltpu.VMEM((CHUNK, DIM), jnp.float32),
        pltpu.VMEM((CHUNK, 2 * DIM), jnp.float32),
        pltpu.VMEM((CHUNK, ACC_W), jnp.float32),
        pltpu.VMEM((ACC_W,), jnp.float32),
        pltpu.VMEM_SHARED((N_PAD, ACC_W), jnp.float32),
        pltpu.SemaphoreType.DMA,
        pltpu.SemaphoreType.DMA,
    ],
)
def _edge_sc(row_hbm, col_hbm, a_hbm, bc_hbm, w2b2_hbm, out_hbm,
             row_v, col_v, a_v, bc_v, m_v, w2_v, acc, sem_a, sem_bc):
    c = lax.axis_index("c")
    s = lax.axis_index("s")
    wid = s * NC + c

    zeros16 = jnp.zeros((16,), jnp.float32)

    def _zrow(e, carry):
        for k in range(ACC_W // 16):
            m_v[e, pl.ds(k * 16, 16)] = zeros16
        return carry

    lax.fori_loop(0, CHUNK, _zrow, 0)

    def _zacc(j, carry):
        pltpu.sync_copy(m_v, acc.at[pl.ds(s * ROWS_PER_SUB + j * CHUNK, CHUNK)])
        return carry

    lax.fori_loop(0, ROWS_PER_SUB // CHUNK, _zacc, 0)

    deg16 = jnp.where(lax.iota(jnp.int32, 16) == 0, 1.0, 0.0).astype(jnp.float32)

    def _deg1(e, carry):
        m_v[e, pl.ds(DIM, 16)] = deg16
        return carry

    lax.fori_loop(0, CHUNK, _deg1, 0)

    pltpu.sync_copy(w2b2_hbm, w2_v)
    w2r = [w2_v[pl.ds(k * 16, 16)] for k in range(DIM // 16)]
    b2s = w2_v[pl.ds(DIM, 16)][0]

    plsc.subcore_barrier()

    ebase = wid * EDGES_PER_W

    def _chunk(i, carry):
        base = ebase + i * CHUNK
        pltpu.sync_copy(row_hbm.at[pl.ds(base, CHUNK)], row_v)
        pltpu.sync_copy(col_hbm.at[pl.ds(base, CHUNK)], col_v)
        ca = pltpu.async_copy(a_hbm.at[row_v], a_v, sem_a)
        cb = pltpu.async_copy(bc_hbm.at[col_v], bc_v, sem_bc)
        ca.wait()
        cb.wait()

        def _edge(e, ecarry):
            parts = []
            for k in range(DIM // 16):
                av = a_v[e, pl.ds(k * 16, 16)]
                bv = bc_v[e, pl.ds(k * 16, 16)]
                parts.append(jnp.maximum(av + bv, 0.0) * w2r[k])
            t0 = (parts[0] + parts[1]) + (parts[2] + parts[3])
            t1 = (parts[4] + parts[5]) + (parts[6] + parts[7])
            dot = jnp.sum(t0 + t1) + b2s
            zv = jnp.broadcast_to(dot, (16,))
            gv = 1.0 / (1.0 + jnp.exp(-zv))
            for k in range(DIM // 16):
                cv = bc_v[e, pl.ds(DIM + k * 16, 16)]
                m_v[e, pl.ds(k * 16, 16)] = gv * cv
            return ecarry

        lax.fori_loop(0, CHUNK, _edge, 0)
        pltpu.sync_copy(m_v, acc.at[row_v], add=True)
        return carry

    lax.fori_loop(0, N_CHUNKS, _chunk, 0)

    plsc.subcore_barrier()

    def _cp(j, carry):
        r0 = s * ROWS_PER_SUB + j * CHUNK
        pltpu.sync_copy(acc.at[pl.ds(r0, CHUNK)], m_v)
        pltpu.sync_copy(m_v, out_hbm.at[c, pl.ds(r0, CHUNK)])
        return carry

    lax.fori_loop(0, ROWS_PER_SUB // CHUNK, _cp, 0)


def _final_body(x_ref, acc_ref, gamma_ref, beta_ref, o_ref):
    sacc = acc_ref[0] + acc_ref[1]
    msg = sacc[:, :DIM]
    deg = jnp.maximum(sacc[:, DIM:DIM + 1], 1.0)
    y = x_ref[...] + msg / deg
    mean = jnp.mean(y, axis=1, keepdims=True)
    cent = y - mean
    var = jnp.mean(cent * cent, axis=1, keepdims=True)
    yn = cent * lax.rsqrt(var + 1e-5)
    o_ref[...] = jnp.maximum(gamma_ref[...] * yn + beta_ref[...], 0.0)


_final = pl.pallas_call(
    _final_body,
    grid=(N // ROW_BLK,),
    in_specs=[
        pl.BlockSpec((ROW_BLK, DIM), lambda i: (i, 0)),
        pl.BlockSpec((NC, ROW_BLK, ACC_W), lambda i: (0, i, 0)),
        pl.BlockSpec((1, DIM), lambda i: (0, 0)),
        pl.BlockSpec((1, DIM), lambda i: (0, 0)),
    ],
    out_specs=pl.BlockSpec((ROW_BLK, DIM), lambda i: (i, 0)),
    out_shape=jax.ShapeDtypeStruct((N, DIM), jnp.float32),
)


def kernel(x, edge_index, W, W1, b1, W2, b2, gamma, beta):
    wh = jnp.concatenate([W1[:DIM], W1[DIM:], W], axis=1)
    bh = jnp.concatenate([jnp.zeros((DIM,), jnp.float32), b1,
                          jnp.zeros((DIM,), jnp.float32)])[None]
    w2b2 = (jnp.zeros((ACC_W,), jnp.float32)
            .at[:DIM].set(W2[:, 0]).at[DIM].set(b2[0]))

    a_arr, bc_arr = _node_mm(x, wh, bh)
    acc = _edge_sc(edge_index[0], edge_index[1], a_arr, bc_arr, w2b2)
    return _final(x, acc[:, :N, :], gamma[None], beta[None])


# speedup vs baseline: 5.6607x; 2.8415x over previous
import functools

import jax
import jax.numpy as jnp
from jax import lax
from jax.experimental import pallas as pl
from jax.experimental.pallas import tpu as pltpu
from jax.experimental.pallas import tpu_sc as plsc

N = 10000
DIM = 128
E = 320000

NC = 2
NS = 16
NW = NC * NS

ACC_W = 144
N_PAD = 10000
ROWS_PER_SUB = N_PAD // NS
CHUNK = 40
EDGES_PER_W = E // NW
N_PAIRS = EDGES_PER_W // (2 * CHUNK)

ROW_BLK = 2000


def _node_mm_body(x_ref, wh_ref, bh_ref, a_ref, bc_ref):
    y = jnp.dot(x_ref[...], wh_ref[...], preferred_element_type=jnp.float32)
    y = y + bh_ref[...]
    a_ref[...] = y[:, :DIM]
    bc_ref[...] = y[:, DIM:]


_node_mm = pl.pallas_call(
    _node_mm_body,
    grid=(N // ROW_BLK,),
    in_specs=[
        pl.BlockSpec((ROW_BLK, DIM), lambda i: (i, 0)),
        pl.BlockSpec((DIM, 3 * DIM), lambda i: (0, 0)),
        pl.BlockSpec((1, 3 * DIM), lambda i: (0, 0)),
    ],
    out_specs=[
        pl.BlockSpec((ROW_BLK, DIM), lambda i: (i, 0)),
        pl.BlockSpec((ROW_BLK, 2 * DIM), lambda i: (i, 0)),
    ],
    out_shape=[
        jax.ShapeDtypeStruct((N, DIM), jnp.float32),
        jax.ShapeDtypeStruct((N, 2 * DIM), jnp.float32),
    ],
)


_sc_mesh = plsc.VectorSubcoreMesh(core_axis_name="c", subcore_axis_name="s")


@functools.partial(
    pl.kernel,
    out_type=jax.ShapeDtypeStruct((NC, N_PAD, ACC_W), jnp.float32),
    mesh=_sc_mesh,
    compiler_params=pltpu.CompilerParams(needs_layout_passes=False,
                                         use_tc_tiling_on_sc=False),
    scratch_types=[
        pltpu.VMEM((CHUNK,), jnp.int32),
        pltpu.VMEM((CHUNK,), jnp.int32),
        pltpu.VMEM((CHUNK,), jnp.int32),
        pltpu.VMEM((CHUNK,), jnp.int32),
        pltpu.VMEM((CHUNK, DIM), jnp.float32),
        pltpu.VMEM((CHUNK, 2 * DIM), jnp.float32),
        pltpu.VMEM((CHUNK, DIM), jnp.float32),
        pltpu.VMEM((CHUNK, 2 * DIM), jnp.float32),
        pltpu.VMEM((CHUNK, ACC_W), jnp.float32),
        pltpu.VMEM((ACC_W,), jnp.float32),
        pltpu.VMEM_SHARED((N_PAD, ACC_W), jnp.float32),
        pltpu.SemaphoreType.DMA,
        pltpu.SemaphoreType.DMA,
        pltpu.SemaphoreType.DMA,
        pltpu.SemaphoreType.DMA,
    ],
)
def _edge_sc(row_hbm, col_hbm, a_hbm, bc_hbm, w2b2_hbm, out_hbm,
             row_v0, col_v0, row_v1, col_v1, a_v0, bc_v0, a_v1, bc_v1,
             m_v, w2_v, acc, sem_a0, sem_b0, sem_a1, sem_b1):
    c = lax.axis_index("c")
    s = lax.axis_index("s")
    wid = s * NC + c

    zeros16 = jnp.zeros((16,), jnp.float32)

    def _zrow(e, carry):
        for k in range(ACC_W // 16):
            m_v[e, pl.ds(k * 16, 16)] = zeros16
        return carry

    lax.fori_loop(0, CHUNK, _zrow, 0)

    def _zacc(j, carry):
        pltpu.sync_copy(m_v.at[pl.ds(0, 25)],
                        acc.at[pl.ds(s * ROWS_PER_SUB + j * 25, 25)])
        return carry

    lax.fori_loop(0, ROWS_PER_SUB // 25, _zacc, 0)

    deg16 = jnp.where(lax.iota(jnp.int32, 16) == 0, 1.0, 0.0).astype(jnp.float32)

    def _deg1(e, carry):
        m_v[e, pl.ds(DIM, 16)] = deg16
        return carry

    lax.fori_loop(0, CHUNK, _deg1, 0)

    pltpu.sync_copy(w2b2_hbm, w2_v)
    w2r = [w2_v[pl.ds(k * 16, 16)] for k in range(DIM // 16)]
    b2s = w2_v[pl.ds(DIM, 16)][0]

    plsc.subcore_barrier()

    ebase = wid * EDGES_PER_W

    def _fetch(base, row_v, col_v, a_v, bc_v, sem_a, sem_b):
        pltpu.sync_copy(row_hbm.at[pl.ds(base, CHUNK)], row_v)
        pltpu.sync_copy(col_hbm.at[pl.ds(base, CHUNK)], col_v)
        pltpu.async_copy(a_hbm.at[row_v], a_v, sem_a)
        pltpu.async_copy(bc_hbm.at[col_v], bc_v, sem_b)

    def _drain(a_v, bc_v, sem_a, sem_b):
        pltpu.make_async_copy(a_hbm.at[pl.ds(0, CHUNK)], a_v, sem_a).wait()
        pltpu.make_async_copy(bc_hbm.at[pl.ds(0, CHUNK)], bc_v, sem_b).wait()

    def _compute(row_v, a_v, bc_v):
        @plsc.parallel_loop(0, CHUNK, 1, unroll=4)
        def _edge(e):
            parts = []
            for k in range(DIM // 16):
                av = a_v[e, pl.ds(k * 16, 16)]
                bv = bc_v[e, pl.ds(k * 16, 16)]
                parts.append(jnp.maximum(av + bv, 0.0) * w2r[k])
            t0 = (parts[0] + parts[1]) + (parts[2] + parts[3])
            t1 = (parts[4] + parts[5]) + (parts[6] + parts[7])
            dot = jnp.sum(t0 + t1) + b2s
            zv = jnp.broadcast_to(dot, (16,))
            gv = 1.0 / (1.0 + jnp.exp(-zv))
            for k in range(DIM // 16):
                cv = bc_v[e, pl.ds(DIM + k * 16, 16)]
                m_v[e, pl.ds(k * 16, 16)] = gv * cv

        pltpu.sync_copy(m_v, acc.at[row_v], add=True)

    _fetch(ebase, row_v0, col_v0, a_v0, bc_v0, sem_a0, sem_b0)

    def _pair(i, carry):
        base1 = ebase + (2 * i + 1) * CHUNK
        _fetch(base1, row_v1, col_v1, a_v1, bc_v1, sem_a1, sem_b1)
        _drain(a_v0, bc_v0, sem_a0, sem_b0)
        _compute(row_v0, a_v0, bc_v0)
        base2 = jnp.minimum(ebase + (2 * i + 2) * CHUNK, E - CHUNK)
        _fetch(base2, row_v0, col_v0, a_v0, bc_v0, sem_a0, sem_b0)
        _drain(a_v1, bc_v1, sem_a1, sem_b1)
        _compute(row_v1, a_v1, bc_v1)
        return carry

    lax.fori_loop(0, N_PAIRS, _pair, 0)
    _drain(a_v0, bc_v0, sem_a0, sem_b0)

    plsc.subcore_barrier()

    def _cp(j, carry):
        r0 = s * ROWS_PER_SUB + j * 25
        pltpu.sync_copy(acc.at[pl.ds(r0, 25)], m_v.at[pl.ds(0, 25)])
        pltpu.sync_copy(m_v.at[pl.ds(0, 25)], out_hbm.at[c, pl.ds(r0, 25)])
        return carry

    lax.fori_loop(0, ROWS_PER_SUB // 25, _cp, 0)


def _final_body(x_ref, acc_ref, gamma_ref, beta_ref, o_ref):
    sacc = acc_ref[0] + acc_ref[1]
    msg = sacc[:, :DIM]
    deg = jnp.maximum(sacc[:, DIM:DIM + 1], 1.0)
    y = x_ref[...] + msg / deg
    mean = jnp.mean(y, axis=1, keepdims=True)
    cent = y - mean
    var = jnp.mean(cent * cent, axis=1, keepdims=True)
    yn = cent * lax.rsqrt(var + 1e-5)
    o_ref[...] = jnp.maximum(gamma_ref[...] * yn + beta_ref[...], 0.0)


_final = pl.pallas_call(
    _final_body,
    grid=(N // ROW_BLK,),
    in_specs=[
        pl.BlockSpec((ROW_BLK, DIM), lambda i: (i, 0)),
        pl.BlockSpec((NC, ROW_BLK, ACC_W), lambda i: (0, i, 0)),
        pl.BlockSpec((1, DIM), lambda i: (0, 0)),
        pl.BlockSpec((1, DIM), lambda i: (0, 0)),
    ],
    out_specs=pl.BlockSpec((ROW_BLK, DIM), lambda i: (i, 0)),
    out_shape=jax.ShapeDtypeStruct((N, DIM), jnp.float32),
)


def kernel(x, edge_index, W, W1, b1, W2, b2, gamma, beta):
    wh = jnp.concatenate([W1[:DIM], W1[DIM:], W], axis=1)
    bh = jnp.concatenate([jnp.zeros((DIM,), jnp.float32), b1,
                          jnp.zeros((DIM,), jnp.float32)])[None]
    w2b2 = (jnp.zeros((ACC_W,), jnp.float32)
            .at[:DIM].set(W2[:, 0]).at[DIM].set(b2[0]))

    a_arr, bc_arr = _node_mm(x, wh, bh)
    acc = _edge_sc(edge_index[0], edge_index[1], a_arr, bc_arr, w2b2)
    return _final(x, acc[:, :N, :], gamma[None], beta[None])
